# Initial kernel scaffold; baseline (speedup 1.0000x reference)
import jax
import jax.numpy as jnp
from jax.experimental import pallas as pl


def kernel(x):
    raise NotImplementedError("write your pallas kernel here")


# speedup vs baseline: 2.5087x; 2.5087x over previous
import functools

import numpy as np
import jax
import jax.numpy as jnp
from jax import lax
from jax.experimental import pallas as pl
from jax.experimental.pallas import tpu as pltpu
from jax.experimental.pallas import tpu_sc as plsc

_B, _T, _HW, _C = 16, 16, 196, 768
_ROWS = _B * _T * _HW
_NW = 32
_RPW = _ROWS // _NW
_CHUNK = 56
_NCHUNK = _RPW // _CHUNK


def _patch_shifts() -> np.ndarray:
    table = np.array([-4, 1, 2, -1, 0, 3, -2, -3, 4])
    inv = 0
    s = np.zeros(_HW, np.int64)
    for idx in range(_HW):
        w, h = idx % 7, idx // 7
        wm, hm = w % 3, h % 3
        if wm == 1 and hm == 1 and w != h:
            inv = -1
        code = wm * 3 + hm
        s[idx] = inv if code == 4 else table[code]
    return s


def _gather_indices() -> np.ndarray:
    s = _patch_shifts()
    b = np.arange(_B)[:, None, None]
    t = np.arange(_T)[None, :, None]
    p = np.arange(_HW)[None, None, :]
    src_t = (t - s[None, None, :]) % _T
    idx = b * (_T * _HW) + src_t * _HW + p
    return idx.reshape(_NW, _NCHUNK, _CHUNK).astype(np.int32)


_IDX = _gather_indices()


@functools.cache
def _build_sc_patch_shift():
    @functools.partial(
        pl.kernel,
        mesh=plsc.VectorSubcoreMesh(core_axis_name="c", subcore_axis_name="s"),
        out_type=jax.ShapeDtypeStruct((_ROWS, _C), jnp.float32),
        scratch_types=[
            pltpu.VMEM((_NCHUNK, _CHUNK), jnp.int32),
            pltpu.VMEM((_CHUNK, _C), jnp.float32),
            pltpu.VMEM((_CHUNK, _C), jnp.float32),
            pltpu.SemaphoreType.DMA,
            pltpu.SemaphoreType.DMA,
        ],
    )
    def _sc_patch_shift(x_hbm, idx_hbm, out_hbm, idx_v, buf0, buf1, gs0, gs1):
        wid = lax.axis_index("s") * 2 + lax.axis_index("c")
        base = wid * _RPW
        pltpu.sync_copy(idx_hbm.at[wid], idx_v)

        def start_gather(c, buf, sem):
            pltpu.async_copy(x_hbm.at[idx_v.at[c]], buf, sem)

        def wait_gather(c, buf, sem):
            pltpu.make_async_copy(x_hbm.at[idx_v.at[c]], buf, sem).wait()

        def scatter(c, buf):
            pltpu.sync_copy(buf, out_hbm.at[pl.ds(base + c * _CHUNK, _CHUNK)])

        start_gather(0, buf0, gs0)
        start_gather(1, buf1, gs1)

        def body(i, carry):
            g = 2 * i
            wait_gather(g, buf0, gs0)
            scatter(g, buf0)
            start_gather(g + 2, buf0, gs0)
            wait_gather(g + 1, buf1, gs1)
            scatter(g + 1, buf1)
            start_gather(g + 3, buf1, gs1)
            return carry

        lax.fori_loop(0, (_NCHUNK - 2) // 2, body, 0)

        g = _NCHUNK - 2
        wait_gather(g, buf0, gs0)
        scatter(g, buf0)
        wait_gather(g + 1, buf1, gs1)
        scatter(g + 1, buf1)

    return _sc_patch_shift


def kernel(x):
    x_flat = x.reshape(_ROWS, _C)
    out = _build_sc_patch_shift()(x_flat, jnp.asarray(_IDX))
    return out.reshape(_B, _T, 14, 14, _C)
